# baseline (speedup 1.0000x reference)
import dataclasses
import functools

import jax
import jax.numpy as jnp
from jax import lax
from jax.experimental import pallas as pl
from jax.experimental.pallas import tpu as pltpu
from jax.experimental.pallas import tpu_sc as plsc

_N = 10000
_E = 320000
_D = 128
_H = 4
_C = 32
_B = 64
_HC = _H * _C

_NTILES = 32
_CHUNK = 48
_NCHUNK = 2 * -(-_E // (_NTILES * _CHUNK * 2))
_EPT = _NCHUNK * _CHUNK
_EPAD = _EPT * _NTILES

_ACCW = 144
_ROWS_PER_TILE = (-(-(_N + 1) // 16) + 7) // 8 * 8
_NROWS = _ROWS_PER_TILE * 16
_TBL = _NROWS


def _lrelu(x, s):
    return jnp.maximum(x, s * x)


def _proj_body(x1, x2, w1l, w1r, w2l, w2r, o1l, o1r, o2l, o2r):
    o1l[...] = jnp.dot(x1[...], w1l[...], preferred_element_type=jnp.float32)
    o1r[...] = jnp.dot(x1[...], w1r[...], preferred_element_type=jnp.float32)
    o2l[...] = jnp.dot(x2[...], w2l[...], preferred_element_type=jnp.float32)
    o2r[...] = jnp.dot(x2[...], w2r[...], preferred_element_type=jnp.float32)


def _project(x1p, x2p, W1l, W1r, W2l, W2r):
    sh = jax.ShapeDtypeStruct((_TBL, _HC), jnp.float32)
    return pl.pallas_call(
        _proj_body,
        out_shape=(sh, sh, sh, sh),
    )(x1p, x2p, W1l, W1r, W2l, W2r)


_LOG2E = 1.4426950408889634
_LN2 = 0.6931471805599453
_RND = 12582912.0


def _exp16(p):
    t = p * _LOG2E
    kf = (t + _RND) - _RND
    g = (t - kf) * _LN2
    r = 1.0 / 5040.0
    for c in (1.0 / 720.0, 1.0 / 120.0, 1.0 / 24.0, 1.0 / 6.0, 0.5, 1.0, 1.0):
        r = r * g + c
    ki = kf.astype(jnp.int32)
    bits = plsc.bitcast(r, jnp.int32) + (ki << 23)
    return plsc.bitcast(bits, jnp.float32)


def _vgather(v, idx):
    return lax.gather(
        v, idx[:, None],
        lax.GatherDimensionNumbers(offset_dims=(), collapsed_slice_dims=(0,),
                                   start_index_map=(0,)),
        slice_sizes=(1,), mode=lax.GatherScatterMode.PROMISE_IN_BOUNDS)


def _edge_body(xl_hbm, xr_hbm, src_hbm, dst_hbm, att_hbm, out_hbm,
               src_v0, dst_v0, src_v1, dst_v1, xl_v0, xr_v0, xl_v1, xr_v1,
               row_v0, row_v1, dvs_v0, dvs_v1, att_v, acc_sh,
               sem0, sem1, ssem0, ssem1):
    row_v = row_v0
    cid = lax.axis_index("c")
    sid = lax.axis_index("s")
    gtid = cid * 16 + sid

    pltpu.sync_copy(att_hbm, att_v)
    att_regs = [(att_v[pl.ds(h * 32, 16)], att_v[pl.ds(h * 32 + 16, 16)])
                for h in range(_H)]
    lane = lax.iota(jnp.int32, 16)
    zero16 = jnp.zeros((16,), jnp.float32)

    @pl.loop(0, _CHUNK)
    def _(r):
        for c in range(_ACCW // 16):
            row_v[r, pl.ds(c * 16, 16)] = zero16

    rbase = sid * _ROWS_PER_TILE
    done = 0
    while done < _ROWS_PER_TILE:
        sz = min(_CHUNK, _ROWS_PER_TILE - done)
        pltpu.sync_copy(row_v.at[pl.ds(0, sz)],
                        acc_sh.at[pl.ds(rbase + done, sz)])
        done += sz
    plsc.subcore_barrier()

    ebase = gtid * _EPT
    bufs = ((src_v0, dst_v0, xl_v0, xr_v0, sem0, row_v0, dvs_v0, ssem0),
            (src_v1, dst_v1, xl_v1, xr_v1, sem1, row_v1, dvs_v1, ssem1))

    def start_gather(ci, b):
        sv, dv, xlv, xrv, sem = bufs[b][:5]
        off = ebase + ci * _CHUNK
        pltpu.sync_copy(src_hbm.at[pl.ds(off, _CHUNK)], sv)
        pltpu.sync_copy(dst_hbm.at[pl.ds(off, _CHUNK)], dv)
        pltpu.async_copy(xl_hbm.at[sv], xlv, sem)
        pltpu.async_copy(xr_hbm.at[dv], xrv, sem)

    def wait_gather(b):
        sv, dv, xlv, xrv, sem = bufs[b][:5]
        pltpu.make_async_copy(xl_hbm.at[sv], xlv, sem).wait()
        pltpu.make_async_copy(xr_hbm.at[dv], xrv, sem).wait()

    def wait_scatter(b):
        _, _, _, _, _, rv, dvs, ssem = bufs[b]
        pltpu.make_async_copy(rv, acc_sh.at[dvs], ssem).wait()

    def compute_scatter(b):
        sv, dv, xlv, xrv, sem, rv, dvs, ssem = bufs[b]

        for c in range(_CHUNK // 16):
            dvs[pl.ds(c * 16, 16)] = dv[pl.ds(c * 16, 16)]

        @plsc.parallel_loop(0, _CHUNK, unroll=4)
        def _(i):
            den = zero16
            for h in range(_H):
                a0 = xlv[i, pl.ds(h * 32, 16)]
                a1 = xlv[i, pl.ds(h * 32 + 16, 16)]
                b0 = xrv[i, pl.ds(h * 32, 16)]
                b1 = xrv[i, pl.ds(h * 32 + 16, 16)]
                s0 = a0 + b0
                s1 = a1 + b1
                e0 = jnp.maximum(s0, 0.2 * s0)
                e1 = jnp.maximum(s1, 0.2 * s1)
                p = e0 * att_regs[h][0] + e1 * att_regs[h][1]
                for sh in (8, 4, 2, 1):
                    p = p + _vgather(p, lane ^ sh)
                ex = jnp.exp(p)
                rv[i, pl.ds(h * 32, 16)] = ex * a0
                rv[i, pl.ds(h * 32 + 16, 16)] = ex * a1
                den = jnp.where(lane == h, ex, den)
            rv[i, pl.ds(_HC, 16)] = den

        pltpu.async_copy(rv, acc_sh.at[dvs], ssem, add=True)

    start_gather(0, 0)

    @pl.loop(0, _NCHUNK // 2)
    def _(j):
        c0 = 2 * j
        start_gather(c0 + 1, 1)
        wait_gather(0)

        @pl.when(j > 0)
        def _():
            wait_scatter(0)

        compute_scatter(0)

        @pl.when(c0 + 2 < _NCHUNK)
        def _():
            start_gather(c0 + 2, 0)

        wait_gather(1)

        @pl.when(j > 0)
        def _():
            wait_scatter(1)

        compute_scatter(1)

    wait_scatter(0)
    wait_scatter(1)
    plsc.subcore_barrier()

    done = 0
    while done < _ROWS_PER_TILE:
        sz = min(_CHUNK, _ROWS_PER_TILE - done)
        pltpu.sync_copy(acc_sh.at[pl.ds(rbase + done, sz)],
                        out_hbm.at[cid, pl.ds(rbase + done, sz)])
        done += sz


def _edge_pass(xl, xr, src, dst, att_flat):
    mesh = plsc.VectorSubcoreMesh(core_axis_name="c", subcore_axis_name="s")
    cp = pltpu.CompilerParams()
    for f, v in (("needs_layout_passes", False),
                 ("use_tc_tiling_on_sc", False)):
        if f in pltpu.CompilerParams.__dataclass_fields__:
            cp = dataclasses.replace(cp, **{f: v})
    k = functools.partial(
        pl.kernel,
        mesh=mesh,
        compiler_params=cp,
        out_type=jax.ShapeDtypeStruct((2, _NROWS, _ACCW), jnp.float32),
        scratch_types=[
            pltpu.VMEM((_CHUNK,), jnp.int32),
            pltpu.VMEM((_CHUNK,), jnp.int32),
            pltpu.VMEM((_CHUNK,), jnp.int32),
            pltpu.VMEM((_CHUNK,), jnp.int32),
            pltpu.VMEM((_CHUNK, _D), jnp.float32),
            pltpu.VMEM((_CHUNK, _D), jnp.float32),
            pltpu.VMEM((_CHUNK, _D), jnp.float32),
            pltpu.VMEM((_CHUNK, _D), jnp.float32),
            pltpu.VMEM((_CHUNK, _ACCW), jnp.float32),
            pltpu.VMEM((_CHUNK, _ACCW), jnp.float32),
            pltpu.VMEM((_CHUNK,), jnp.int32),
            pltpu.VMEM((_CHUNK,), jnp.int32),
            pltpu.VMEM((_D,), jnp.float32),
            pltpu.VMEM_SHARED((_NROWS, _ACCW), jnp.float32),
            pltpu.SemaphoreType.DMA,
            pltpu.SemaphoreType.DMA,
            pltpu.SemaphoreType.DMA,
            pltpu.SemaphoreType.DMA,
        ],
    )(_edge_body)
    return k(xl, xr, src, dst, att_flat)


_RBLK = 1000
_NBLK = _N // _RBLK


def _head_body(acc1, acc2, bat1, bat2, b1g, b2g, p1W, p1b, p2W, p2b,
               f1W, f1b, f2W, f2b, oW, ob, out,
               pooled1, pooled2, cnt1, cnt2):
    i = pl.program_id(0)

    @pl.when(i == 0)
    def _():
        pooled1[...] = jnp.zeros_like(pooled1)
        pooled2[...] = jnp.zeros_like(pooled2)
        cnt1[...] = jnp.zeros_like(cnt1)
        cnt2[...] = jnp.zeros_like(cnt2)

    iota_b = lax.broadcasted_iota(jnp.int32, (1, _B), 1)
    ones_col = jnp.ones((_RBLK, 1), jnp.float32)

    for acc, bat, bg, pooled, cnt in (
            (acc1, bat1, b1g, pooled1, cnt1),
            (acc2, bat2, b2g, pooled2, cnt2)):
        blk = acc[...]
        comb = blk[0] + blk[1]
        feats = []
        for h in range(_H):
            d = comb[:, 128 + h][:, None]
            feats.append(comb[:, h * 32:(h + 1) * 32] / (d + 1e-16))
        feat = jnp.concatenate(feats, axis=1) + bg[...][None, :]
        feat = _lrelu(feat, 0.01)
        onehot = (bat[...].reshape(_RBLK, 1) == iota_b).astype(jnp.float32)
        pooled[...] += lax.dot_general(
            onehot, feat, (((0,), (0,)), ((), ())),
            preferred_element_type=jnp.float32, precision=lax.Precision.HIGHEST)
        cnt[...] += lax.dot_general(
            onehot, ones_col, (((0,), (0,)), ((), ())),
            preferred_element_type=jnp.float32, precision=lax.Precision.HIGHEST)

    @pl.when(i == _NBLK - 1)
    def _():
        m1 = pooled1[...] / jnp.maximum(cnt1[...], 1.0)
        m2 = pooled2[...] / jnp.maximum(cnt2[...], 1.0)
        x1 = _lrelu(jnp.dot(m1, p1W[...], preferred_element_type=jnp.float32)
                    + p1b[...][None, :], 0.01)
        x2 = _lrelu(jnp.dot(m2, p2W[...], preferred_element_type=jnp.float32)
                    + p2b[...][None, :], 0.01)
        xc = jnp.concatenate([x1, x2], axis=1)
        hc = _lrelu(jnp.dot(xc, f1W[...], preferred_element_type=jnp.float32)
                    + f1b[...][None, :], 0.01)
        h2 = _lrelu(jnp.dot(hc, f2W[...], preferred_element_type=jnp.float32)
                    + f2b[...][None, :], 0.01)
        out[...] = (jnp.dot(h2, oW[...], preferred_element_type=jnp.float32)
                    + ob[...][None, :])


def _head(acc1, acc2, bat1, bat2, b1g, b2g, p1W, p1b, p2W, p2b,
          f1W, f1b, f2W, f2b, oW, ob):
    full = lambda shape: pl.BlockSpec(shape, lambda i: tuple(0 for _ in shape))
    return pl.pallas_call(
        _head_body,
        grid=(_NBLK,),
        in_specs=[
            pl.BlockSpec((2, _RBLK, _ACCW), lambda i: (0, i, 0)),
            pl.BlockSpec((2, _RBLK, _ACCW), lambda i: (0, i, 0)),
            pl.BlockSpec((1, 1, _RBLK), lambda i: (i, 0, 0)),
            pl.BlockSpec((1, 1, _RBLK), lambda i: (i, 0, 0)),
            full((_HC,)), full((_HC,)),
            full((_HC, _HC)), full((_HC,)),
            full((_HC, _HC)), full((_HC,)),
            full((2 * _HC, _HC)), full((_HC,)),
            full((_HC, 16)), full((16,)),
            full((16, 1)), full((1,)),
        ],
        out_specs=pl.BlockSpec((_B, 1), lambda i: (0, 0)),
        out_shape=jax.ShapeDtypeStruct((_B, 1), jnp.float32),
        scratch_shapes=[
            pltpu.VMEM((_B, _HC), jnp.float32),
            pltpu.VMEM((_B, _HC), jnp.float32),
            pltpu.VMEM((_B, 1), jnp.float32),
            pltpu.VMEM((_B, 1), jnp.float32),
        ],
    )(acc1, acc2, bat1, bat2, b1g, b2g, p1W, p1b, p2W, p2b,
      f1W, f1b, f2W, f2b, oW, ob)


def kernel(ch1_x, ch1_edge_index, ch1_batch, ch2_x, ch2_edge_index, ch2_batch,
           W1l, W1r, att1, b1g, p1W, p1b, W2l, W2r, att2, b2g, p2W, p2b,
           f1W, f1b, f2W, f2b, oW, ob):
    f32 = jnp.float32
    x1p = jnp.pad(ch1_x.astype(f32), ((0, _TBL - _N), (0, 0)))
    x2p = jnp.pad(ch2_x.astype(f32), ((0, _TBL - _N), (0, 0)))

    xl1, xr1, xl2, xr2 = _project(x1p, x2p, W1l, W1r, W2l, W2r)

    def prep_edges(ei):
        src = jnp.concatenate(
            [ei[0].astype(jnp.int32), jnp.zeros((_EPAD - _E,), jnp.int32)])
        dst = jnp.concatenate(
            [ei[1].astype(jnp.int32), jnp.full((_EPAD - _E,), _N, jnp.int32)])
        return src, dst

    src1, dst1 = prep_edges(ch1_edge_index)
    src2, dst2 = prep_edges(ch2_edge_index)

    acc1 = _edge_pass(xl1, xr1, src1, dst1, att1.reshape(_HC).astype(f32))
    acc2 = _edge_pass(xl2, xr2, src2, dst2, att2.reshape(_HC).astype(f32))

    bat1 = ch1_batch.astype(jnp.int32).reshape(_NBLK, 1, _RBLK)
    bat2 = ch2_batch.astype(jnp.int32).reshape(_NBLK, 1, _RBLK)

    return _head(acc1, acc2, bat1, bat2, b1g, b2g, p1W, p1b, p2W, p2b,
                 f1W, f1b, f2W, f2b, oW, ob)


# speedup vs baseline: 1.1815x; 1.1815x over previous
import dataclasses
import functools

import jax
import jax.numpy as jnp
from jax import lax
from jax.experimental import pallas as pl
from jax.experimental.pallas import tpu as pltpu
from jax.experimental.pallas import tpu_sc as plsc

_N = 10000
_E = 320000
_D = 128
_H = 4
_C = 32
_B = 64
_HC = _H * _C

_NTILES = 32
_CHUNK = 48
_SUPER = 6
_NSUP = -(-_E // (_NTILES * _CHUNK * _SUPER))
_NCHUNK = _NSUP * _SUPER
_EPT = _NCHUNK * _CHUNK
_EPAD = _EPT * _NTILES
_SLEN = _SUPER * _CHUNK

_ACCW = 144
_ROWS_PER_TILE = (-(-(_N + 1) // 16) + 7) // 8 * 8
_NROWS = _ROWS_PER_TILE * 16
_TBL = _NROWS


def _lrelu(x, s):
    return jnp.maximum(x, s * x)


def _proj_body(x1, x2, w1l, w1r, w2l, w2r, o1l, o1r, o2l, o2r):
    o1l[...] = jnp.dot(x1[...], w1l[...], preferred_element_type=jnp.float32)
    o1r[...] = jnp.dot(x1[...], w1r[...], preferred_element_type=jnp.float32)
    o2l[...] = jnp.dot(x2[...], w2l[...], preferred_element_type=jnp.float32)
    o2r[...] = jnp.dot(x2[...], w2r[...], preferred_element_type=jnp.float32)


def _project(x1p, x2p, W1l, W1r, W2l, W2r):
    sh = jax.ShapeDtypeStruct((_TBL, _HC), jnp.float32)
    return pl.pallas_call(
        _proj_body,
        out_shape=(sh, sh, sh, sh),
    )(x1p, x2p, W1l, W1r, W2l, W2r)


_LOG2E = 1.4426950408889634
_LN2 = 0.6931471805599453
_RND = 12582912.0


def _exp16(p):
    t = p * _LOG2E
    kf = (t + _RND) - _RND
    g = (t - kf) * _LN2
    r = 1.0 / 5040.0
    for c in (1.0 / 720.0, 1.0 / 120.0, 1.0 / 24.0, 1.0 / 6.0, 0.5, 1.0, 1.0):
        r = r * g + c
    ki = kf.astype(jnp.int32)
    bits = plsc.bitcast(r, jnp.int32) + (ki << 23)
    return plsc.bitcast(bits, jnp.float32)


def _vgather(v, idx):
    return lax.gather(
        v, idx[:, None],
        lax.GatherDimensionNumbers(offset_dims=(), collapsed_slice_dims=(0,),
                                   start_index_map=(0,)),
        slice_sizes=(1,), mode=lax.GatherScatterMode.PROMISE_IN_BOUNDS)


def _edge_body(xl_hbm, xr_hbm, src_hbm, dst_hbm, att_hbm, out_hbm,
               src_sb, dst_sb, xl_v0, xr_v0, xl_v1, xr_v1,
               row_v0, row_v1, dvs_v0, dvs_v1, att_v, acc_sh,
               sem0, sem1, ssem0, ssem1):
    row_v = row_v0
    cid = lax.axis_index("c")
    sid = lax.axis_index("s")
    gtid = cid * 16 + sid

    pltpu.sync_copy(att_hbm, att_v)
    att_regs = [(att_v[pl.ds(h * 32, 16)], att_v[pl.ds(h * 32 + 16, 16)])
                for h in range(_H)]
    lane = lax.iota(jnp.int32, 16)
    zero16 = jnp.zeros((16,), jnp.float32)

    @pl.loop(0, _CHUNK)
    def _(r):
        for c in range(_ACCW // 16):
            row_v[r, pl.ds(c * 16, 16)] = zero16

    rbase = sid * _ROWS_PER_TILE
    done = 0
    while done < _ROWS_PER_TILE:
        sz = min(_CHUNK, _ROWS_PER_TILE - done)
        pltpu.sync_copy(row_v.at[pl.ds(0, sz)],
                        acc_sh.at[pl.ds(rbase + done, sz)])
        done += sz
    plsc.subcore_barrier()

    ebase = gtid * _EPT
    bufs = ((xl_v0, xr_v0, sem0, row_v0, dvs_v0, ssem0),
            (xl_v1, xr_v1, sem1, row_v1, dvs_v1, ssem1))

    cbase = gtid * _NCHUNK

    def load_super(s):
        off = cbase + s * _SUPER
        pltpu.sync_copy(src_hbm.at[pl.ds(off, _SUPER)], src_sb)
        pltpu.sync_copy(dst_hbm.at[pl.ds(off, _SUPER)], dst_sb)

    def start_gather(ci, b):
        xlv, xrv, sem = bufs[b][:3]
        ciw = lax.rem(ci, _SUPER)
        pltpu.async_copy(xl_hbm.at[src_sb.at[ciw]], xlv, sem)
        pltpu.async_copy(xr_hbm.at[dst_sb.at[ciw]], xrv, sem)

    def wait_gather(b):
        xlv, xrv, sem = bufs[b][:3]
        pltpu.make_async_copy(xl_hbm.at[src_sb.at[0]], xlv, sem).wait()
        pltpu.make_async_copy(xr_hbm.at[dst_sb.at[0]], xrv, sem).wait()

    def wait_scatter(b):
        _, _, _, rv, dvs, ssem = bufs[b]
        pltpu.make_async_copy(rv, acc_sh.at[dvs], ssem).wait()

    def stage_dvs(ci, b):
        dvs = bufs[b][4]
        ciw = lax.rem(ci, _SUPER)
        for c in range(_CHUNK // 16):
            dvs[pl.ds(c * 16, 16)] = dst_sb[ciw, pl.ds(c * 16, 16)]

    def compute_scatter(b):
        xlv, xrv, sem, rv, dvs, ssem = bufs[b]

        @plsc.parallel_loop(0, _CHUNK, unroll=2)
        def _(i):
            den = zero16
            for h in range(_H):
                a0 = xlv[i, pl.ds(h * 32, 16)]
                a1 = xlv[i, pl.ds(h * 32 + 16, 16)]
                b0 = xrv[i, pl.ds(h * 32, 16)]
                b1 = xrv[i, pl.ds(h * 32 + 16, 16)]
                s0 = a0 + b0
                s1 = a1 + b1
                e0 = jnp.maximum(s0, 0.2 * s0)
                e1 = jnp.maximum(s1, 0.2 * s1)
                p = e0 * att_regs[h][0] + e1 * att_regs[h][1]
                for sh in (8, 4, 2, 1):
                    p = p + _vgather(p, lane ^ sh)
                ex = jnp.exp(p)
                rv[i, pl.ds(h * 32, 16)] = ex * a0
                rv[i, pl.ds(h * 32 + 16, 16)] = ex * a1
                den = jnp.where(lane == h, ex, den)
            rv[i, pl.ds(_HC, 16)] = den

        pltpu.async_copy(rv, acc_sh.at[dvs], ssem, add=True)

    load_super(0)
    start_gather(0, 0)

    @pl.loop(0, _NCHUNK // 2)
    def _(j):
        c0 = 2 * j
        boundary = lax.rem(c0 + 2, _SUPER) == 0
        start_gather(c0 + 1, 1)
        wait_gather(0)

        @pl.when(j > 0)
        def _():
            wait_scatter(0)

        stage_dvs(c0, 0)
        compute_scatter(0)

        @pl.when(jnp.logical_and(jnp.logical_not(boundary),
                                 c0 + 2 < _NCHUNK))
        def _():
            start_gather(c0 + 2, 0)

        wait_gather(1)

        @pl.when(j > 0)
        def _():
            wait_scatter(1)

        stage_dvs(c0 + 1, 1)

        @pl.when(jnp.logical_and(boundary, c0 + 2 < _NCHUNK))
        def _():
            load_super((c0 + 2) // _SUPER)
            start_gather(c0 + 2, 0)

        compute_scatter(1)

    wait_scatter(0)
    wait_scatter(1)
    plsc.subcore_barrier()

    done = 0
    while done < _ROWS_PER_TILE:
        sz = min(_CHUNK, _ROWS_PER_TILE - done)
        pltpu.sync_copy(acc_sh.at[pl.ds(rbase + done, sz)],
                        out_hbm.at[cid, pl.ds(rbase + done, sz)])
        done += sz


def _edge_pass(xl, xr, src, dst, att_flat):
    mesh = plsc.VectorSubcoreMesh(core_axis_name="c", subcore_axis_name="s")
    cp = pltpu.CompilerParams()
    for f, v in (("needs_layout_passes", False),
                 ("use_tc_tiling_on_sc", False)):
        if f in pltpu.CompilerParams.__dataclass_fields__:
            cp = dataclasses.replace(cp, **{f: v})
    k = functools.partial(
        pl.kernel,
        mesh=mesh,
        compiler_params=cp,
        out_type=jax.ShapeDtypeStruct((2, _NROWS, _ACCW), jnp.float32),
        scratch_types=[
            pltpu.VMEM((_SUPER, _CHUNK), jnp.int32),
            pltpu.VMEM((_SUPER, _CHUNK), jnp.int32),
            pltpu.VMEM((_CHUNK, _D), jnp.float32),
            pltpu.VMEM((_CHUNK, _D), jnp.float32),
            pltpu.VMEM((_CHUNK, _D), jnp.float32),
            pltpu.VMEM((_CHUNK, _D), jnp.float32),
            pltpu.VMEM((_CHUNK, _ACCW), jnp.float32),
            pltpu.VMEM((_CHUNK, _ACCW), jnp.float32),
            pltpu.VMEM((_CHUNK,), jnp.int32),
            pltpu.VMEM((_CHUNK,), jnp.int32),
            pltpu.VMEM((_D,), jnp.float32),
            pltpu.VMEM_SHARED((_NROWS, _ACCW), jnp.float32),
            pltpu.SemaphoreType.DMA,
            pltpu.SemaphoreType.DMA,
            pltpu.SemaphoreType.DMA,
            pltpu.SemaphoreType.DMA,
        ],
    )(_edge_body)
    return k(xl, xr, src, dst, att_flat)


_RBLK = 1000
_NBLK = _N // _RBLK


def _head_body(acc1, acc2, bat1, bat2, b1g, b2g, p1W, p1b, p2W, p2b,
               f1W, f1b, f2W, f2b, oW, ob, out,
               pooled1, pooled2, cnt1, cnt2):
    i = pl.program_id(0)

    @pl.when(i == 0)
    def _():
        pooled1[...] = jnp.zeros_like(pooled1)
        pooled2[...] = jnp.zeros_like(pooled2)
        cnt1[...] = jnp.zeros_like(cnt1)
        cnt2[...] = jnp.zeros_like(cnt2)

    iota_b = lax.broadcasted_iota(jnp.int32, (1, _B), 1)
    ones_col = jnp.ones((_RBLK, 1), jnp.float32)

    for acc, bat, bg, pooled, cnt in (
            (acc1, bat1, b1g, pooled1, cnt1),
            (acc2, bat2, b2g, pooled2, cnt2)):
        blk = acc[...]
        comb = blk[0] + blk[1]
        feats = []
        for h in range(_H):
            d = comb[:, 128 + h][:, None]
            feats.append(comb[:, h * 32:(h + 1) * 32] / (d + 1e-16))
        feat = jnp.concatenate(feats, axis=1) + bg[...][None, :]
        feat = _lrelu(feat, 0.01)
        onehot = (bat[...].reshape(_RBLK, 1) == iota_b).astype(jnp.float32)
        pooled[...] += lax.dot_general(
            onehot, feat, (((0,), (0,)), ((), ())),
            preferred_element_type=jnp.float32, precision=lax.Precision.HIGHEST)
        cnt[...] += lax.dot_general(
            onehot, ones_col, (((0,), (0,)), ((), ())),
            preferred_element_type=jnp.float32, precision=lax.Precision.HIGHEST)

    @pl.when(i == _NBLK - 1)
    def _():
        m1 = pooled1[...] / jnp.maximum(cnt1[...], 1.0)
        m2 = pooled2[...] / jnp.maximum(cnt2[...], 1.0)
        x1 = _lrelu(jnp.dot(m1, p1W[...], preferred_element_type=jnp.float32)
                    + p1b[...][None, :], 0.01)
        x2 = _lrelu(jnp.dot(m2, p2W[...], preferred_element_type=jnp.float32)
                    + p2b[...][None, :], 0.01)
        xc = jnp.concatenate([x1, x2], axis=1)
        hc = _lrelu(jnp.dot(xc, f1W[...], preferred_element_type=jnp.float32)
                    + f1b[...][None, :], 0.01)
        h2 = _lrelu(jnp.dot(hc, f2W[...], preferred_element_type=jnp.float32)
                    + f2b[...][None, :], 0.01)
        out[...] = (jnp.dot(h2, oW[...], preferred_element_type=jnp.float32)
                    + ob[...][None, :])


def _head(acc1, acc2, bat1, bat2, b1g, b2g, p1W, p1b, p2W, p2b,
          f1W, f1b, f2W, f2b, oW, ob):
    full = lambda shape: pl.BlockSpec(shape, lambda i: tuple(0 for _ in shape))
    return pl.pallas_call(
        _head_body,
        grid=(_NBLK,),
        in_specs=[
            pl.BlockSpec((2, _RBLK, _ACCW), lambda i: (0, i, 0)),
            pl.BlockSpec((2, _RBLK, _ACCW), lambda i: (0, i, 0)),
            pl.BlockSpec((1, 1, _RBLK), lambda i: (i, 0, 0)),
            pl.BlockSpec((1, 1, _RBLK), lambda i: (i, 0, 0)),
            full((_HC,)), full((_HC,)),
            full((_HC, _HC)), full((_HC,)),
            full((_HC, _HC)), full((_HC,)),
            full((2 * _HC, _HC)), full((_HC,)),
            full((_HC, 16)), full((16,)),
            full((16, 1)), full((1,)),
        ],
        out_specs=pl.BlockSpec((_B, 1), lambda i: (0, 0)),
        out_shape=jax.ShapeDtypeStruct((_B, 1), jnp.float32),
        scratch_shapes=[
            pltpu.VMEM((_B, _HC), jnp.float32),
            pltpu.VMEM((_B, _HC), jnp.float32),
            pltpu.VMEM((_B, 1), jnp.float32),
            pltpu.VMEM((_B, 1), jnp.float32),
        ],
    )(acc1, acc2, bat1, bat2, b1g, b2g, p1W, p1b, p2W, p2b,
      f1W, f1b, f2W, f2b, oW, ob)


def kernel(ch1_x, ch1_edge_index, ch1_batch, ch2_x, ch2_edge_index, ch2_batch,
           W1l, W1r, att1, b1g, p1W, p1b, W2l, W2r, att2, b2g, p2W, p2b,
           f1W, f1b, f2W, f2b, oW, ob):
    f32 = jnp.float32
    x1p = jnp.pad(ch1_x.astype(f32), ((0, _TBL - _N), (0, 0)))
    x2p = jnp.pad(ch2_x.astype(f32), ((0, _TBL - _N), (0, 0)))

    xl1, xr1, xl2, xr2 = _project(x1p, x2p, W1l, W1r, W2l, W2r)

    def prep_edges(ei):
        src = jnp.concatenate(
            [ei[0].astype(jnp.int32), jnp.zeros((_EPAD - _E,), jnp.int32)])
        dst = jnp.concatenate(
            [ei[1].astype(jnp.int32), jnp.full((_EPAD - _E,), _N, jnp.int32)])
        return (src.reshape(_EPAD // _CHUNK, _CHUNK),
                dst.reshape(_EPAD // _CHUNK, _CHUNK))

    src1, dst1 = prep_edges(ch1_edge_index)
    src2, dst2 = prep_edges(ch2_edge_index)

    acc1 = _edge_pass(xl1, xr1, src1, dst1, att1.reshape(_HC).astype(f32))
    acc2 = _edge_pass(xl2, xr2, src2, dst2, att2.reshape(_HC).astype(f32))

    bat1 = ch1_batch.astype(jnp.int32).reshape(_NBLK, 1, _RBLK)
    bat2 = ch2_batch.astype(jnp.int32).reshape(_NBLK, 1, _RBLK)

    return _head(acc1, acc2, bat1, bat2, b1g, b2g, p1W, p1b, p2W, p2b,
                 f1W, f1b, f2W, f2b, oW, ob)


# speedup vs baseline: 1.2396x; 1.0492x over previous
import dataclasses
import functools

import jax
import jax.numpy as jnp
from jax import lax
from jax.experimental import pallas as pl
from jax.experimental.pallas import tpu as pltpu
from jax.experimental.pallas import tpu_sc as plsc

_N = 10000
_E = 320000
_D = 128
_H = 4
_C = 32
_B = 64
_HC = _H * _C

_NTILES = 32
_CHUNK = 48
_SUPER = 6
_NSUP = -(-_E // (_NTILES * _CHUNK * _SUPER))
_NCHUNK = _NSUP * _SUPER
_NCHUNK0 = _NCHUNK + 18
_NCHUNK1 = 2 * _NCHUNK - _NCHUNK0
_EPT0 = _NCHUNK0 * _CHUNK
_EPT1 = _NCHUNK1 * _CHUNK
_EPAD = 16 * (_EPT0 + _EPT1)
_SLEN = _SUPER * _CHUNK

_ACCW = 144
_ROWS_PER_TILE = (-(-(_N + 1) // 16) + 7) // 8 * 8
_NROWS = _ROWS_PER_TILE * 16
_TBL = _NROWS


def _lrelu(x, s):
    return jnp.maximum(x, s * x)


def _proj_body(x1, x2, w1l, w1r, w2l, w2r, o1l, o1r, o2l, o2r):
    o1l[...] = jnp.dot(x1[...], w1l[...], preferred_element_type=jnp.float32)
    o1r[...] = jnp.dot(x1[...], w1r[...], preferred_element_type=jnp.float32)
    o2l[...] = jnp.dot(x2[...], w2l[...], preferred_element_type=jnp.float32)
    o2r[...] = jnp.dot(x2[...], w2r[...], preferred_element_type=jnp.float32)


def _project(x1p, x2p, W1l, W1r, W2l, W2r):
    sh = jax.ShapeDtypeStruct((_TBL, _HC), jnp.float32)
    return pl.pallas_call(
        _proj_body,
        out_shape=(sh, sh, sh, sh),
    )(x1p, x2p, W1l, W1r, W2l, W2r)


_LOG2E = 1.4426950408889634
_LN2 = 0.6931471805599453
_RND = 12582912.0


def _exp16(p):
    t = p * _LOG2E
    kf = (t + _RND) - _RND
    g = (t - kf) * _LN2
    r = 1.0 / 5040.0
    for c in (1.0 / 720.0, 1.0 / 120.0, 1.0 / 24.0, 1.0 / 6.0, 0.5, 1.0, 1.0):
        r = r * g + c
    ki = kf.astype(jnp.int32)
    bits = plsc.bitcast(r, jnp.int32) + (ki << 23)
    return plsc.bitcast(bits, jnp.float32)


def _vgather(v, idx):
    return lax.gather(
        v, idx[:, None],
        lax.GatherDimensionNumbers(offset_dims=(), collapsed_slice_dims=(0,),
                                   start_index_map=(0,)),
        slice_sizes=(1,), mode=lax.GatherScatterMode.PROMISE_IN_BOUNDS)


def _edge_body(xl_hbm, xr_hbm, src_hbm, dst_hbm, att_hbm, out_hbm,
               src_sb, dst_sb, xl_v0, xr_v0, xl_v1, xr_v1,
               row_v0, row_v1, dvs_v0, dvs_v1, att_v, acc_sh,
               sem0, sem1, ssem0, ssem1):
    row_v = row_v0
    cid = lax.axis_index("c")
    sid = lax.axis_index("s")
    gtid = cid * 16 + sid

    pltpu.sync_copy(att_hbm, att_v)
    att_regs = [(att_v[pl.ds(h * 32, 16)], att_v[pl.ds(h * 32 + 16, 16)])
                for h in range(_H)]
    lane = lax.iota(jnp.int32, 16)
    zero16 = jnp.zeros((16,), jnp.float32)

    @pl.loop(0, _CHUNK)
    def _(r):
        for c in range(_ACCW // 16):
            row_v[r, pl.ds(c * 16, 16)] = zero16

    rbase = sid * _ROWS_PER_TILE
    done = 0
    while done < _ROWS_PER_TILE:
        sz = min(_CHUNK, _ROWS_PER_TILE - done)
        pltpu.sync_copy(row_v.at[pl.ds(0, sz)],
                        acc_sh.at[pl.ds(rbase + done, sz)])
        done += sz
    plsc.subcore_barrier()

    nch2 = jnp.where(cid == 0, _NCHUNK0 // 2, _NCHUNK1 // 2)
    nchunk = nch2 * 2
    cbase = jnp.where(cid == 0, sid * _NCHUNK0,
                      16 * _NCHUNK0 + sid * _NCHUNK1)
    bufs = ((xl_v0, xr_v0, sem0, row_v0, dvs_v0, ssem0),
            (xl_v1, xr_v1, sem1, row_v1, dvs_v1, ssem1))

    def load_super(s):
        off = cbase + s * _SUPER
        pltpu.sync_copy(src_hbm.at[pl.ds(off, _SUPER)], src_sb)
        pltpu.sync_copy(dst_hbm.at[pl.ds(off, _SUPER)], dst_sb)

    def start_gather(ci, b):
        xlv, xrv, sem = bufs[b][:3]
        ciw = lax.rem(ci, _SUPER)
        pltpu.async_copy(xl_hbm.at[src_sb.at[ciw]], xlv, sem)
        pltpu.async_copy(xr_hbm.at[dst_sb.at[ciw]], xrv, sem)

    def wait_gather(b):
        xlv, xrv, sem = bufs[b][:3]
        pltpu.make_async_copy(xl_hbm.at[src_sb.at[0]], xlv, sem).wait()
        pltpu.make_async_copy(xr_hbm.at[dst_sb.at[0]], xrv, sem).wait()

    def wait_scatter(b):
        _, _, _, rv, dvs, ssem = bufs[b]
        pltpu.make_async_copy(rv, acc_sh.at[dvs], ssem).wait()

    def stage_dvs(ci, b):
        dvs = bufs[b][4]
        ciw = lax.rem(ci, _SUPER)
        for c in range(_CHUNK // 16):
            dvs[pl.ds(c * 16, 16)] = dst_sb[ciw, pl.ds(c * 16, 16)]

    def compute_scatter(b):
        xlv, xrv, sem, rv, dvs, ssem = bufs[b]

        @plsc.parallel_loop(0, _CHUNK, unroll=2)
        def _(i):
            den = zero16
            for h in range(_H):
                a0 = xlv[i, pl.ds(h * 32, 16)]
                a1 = xlv[i, pl.ds(h * 32 + 16, 16)]
                b0 = xrv[i, pl.ds(h * 32, 16)]
                b1 = xrv[i, pl.ds(h * 32 + 16, 16)]
                s0 = a0 + b0
                s1 = a1 + b1
                e0 = jnp.maximum(s0, 0.2 * s0)
                e1 = jnp.maximum(s1, 0.2 * s1)
                p = e0 * att_regs[h][0] + e1 * att_regs[h][1]
                for sh in (8, 4, 2, 1):
                    p = p + _vgather(p, lane ^ sh)
                ex = jnp.exp(p)
                rv[i, pl.ds(h * 32, 16)] = ex * a0
                rv[i, pl.ds(h * 32 + 16, 16)] = ex * a1
                den = jnp.where(lane == h, ex, den)
            rv[i, pl.ds(_HC, 16)] = den

        pltpu.async_copy(rv, acc_sh.at[dvs], ssem, add=True)

    load_super(0)
    start_gather(0, 0)

    @pl.loop(0, nch2)
    def _(j):
        c0 = 2 * j
        boundary = lax.rem(c0 + 2, _SUPER) == 0
        start_gather(c0 + 1, 1)
        wait_gather(0)

        @pl.when(j > 0)
        def _():
            wait_scatter(0)

        stage_dvs(c0, 0)
        compute_scatter(0)

        @pl.when(jnp.logical_and(jnp.logical_not(boundary),
                                 c0 + 2 < nchunk))
        def _():
            start_gather(c0 + 2, 0)

        wait_gather(1)

        @pl.when(j > 0)
        def _():
            wait_scatter(1)

        stage_dvs(c0 + 1, 1)

        @pl.when(jnp.logical_and(boundary, c0 + 2 < nchunk))
        def _():
            load_super((c0 + 2) // _SUPER)
            start_gather(c0 + 2, 0)

        compute_scatter(1)

    wait_scatter(0)
    wait_scatter(1)
    plsc.subcore_barrier()

    done = 0
    while done < _ROWS_PER_TILE:
        sz = min(_CHUNK, _ROWS_PER_TILE - done)
        pltpu.sync_copy(acc_sh.at[pl.ds(rbase + done, sz)],
                        out_hbm.at[cid, pl.ds(rbase + done, sz)])
        done += sz


def _edge_pass(xl, xr, src, dst, att_flat):
    mesh = plsc.VectorSubcoreMesh(core_axis_name="c", subcore_axis_name="s")
    cp = pltpu.CompilerParams()
    for f, v in (("needs_layout_passes", False),
                 ("use_tc_tiling_on_sc", False)):
        if f in pltpu.CompilerParams.__dataclass_fields__:
            cp = dataclasses.replace(cp, **{f: v})
    k = functools.partial(
        pl.kernel,
        mesh=mesh,
        compiler_params=cp,
        out_type=jax.ShapeDtypeStruct((2, _NROWS, _ACCW), jnp.float32),
        scratch_types=[
            pltpu.VMEM((_SUPER, _CHUNK), jnp.int32),
            pltpu.VMEM((_SUPER, _CHUNK), jnp.int32),
            pltpu.VMEM((_CHUNK, _D), jnp.float32),
            pltpu.VMEM((_CHUNK, _D), jnp.float32),
            pltpu.VMEM((_CHUNK, _D), jnp.float32),
            pltpu.VMEM((_CHUNK, _D), jnp.float32),
            pltpu.VMEM((_CHUNK, _ACCW), jnp.float32),
            pltpu.VMEM((_CHUNK, _ACCW), jnp.float32),
            pltpu.VMEM((_CHUNK,), jnp.int32),
            pltpu.VMEM((_CHUNK,), jnp.int32),
            pltpu.VMEM((_D,), jnp.float32),
            pltpu.VMEM_SHARED((_NROWS, _ACCW), jnp.float32),
            pltpu.SemaphoreType.DMA,
            pltpu.SemaphoreType.DMA,
            pltpu.SemaphoreType.DMA,
            pltpu.SemaphoreType.DMA,
        ],
    )(_edge_body)
    return k(xl, xr, src, dst, att_flat)


_RBLK = 1000
_NBLK = _N // _RBLK


def _head_body(acc1, acc2, bat1, bat2, b1g, b2g, p1W, p1b, p2W, p2b,
               f1W, f1b, f2W, f2b, oW, ob, out,
               pooled1, pooled2, cnt1, cnt2):
    i = pl.program_id(0)

    @pl.when(i == 0)
    def _():
        pooled1[...] = jnp.zeros_like(pooled1)
        pooled2[...] = jnp.zeros_like(pooled2)
        cnt1[...] = jnp.zeros_like(cnt1)
        cnt2[...] = jnp.zeros_like(cnt2)

    iota_b = lax.broadcasted_iota(jnp.int32, (1, _B), 1)
    ones_col = jnp.ones((_RBLK, 1), jnp.float32)

    for acc, bat, bg, pooled, cnt in (
            (acc1, bat1, b1g, pooled1, cnt1),
            (acc2, bat2, b2g, pooled2, cnt2)):
        blk = acc[...]
        comb = blk[0] + blk[1]
        feats = []
        for h in range(_H):
            d = comb[:, 128 + h][:, None]
            feats.append(comb[:, h * 32:(h + 1) * 32] / (d + 1e-16))
        feat = jnp.concatenate(feats, axis=1) + bg[...][None, :]
        feat = _lrelu(feat, 0.01)
        onehot = (bat[...].reshape(_RBLK, 1) == iota_b).astype(jnp.float32)
        pooled[...] += lax.dot_general(
            onehot, feat, (((0,), (0,)), ((), ())),
            preferred_element_type=jnp.float32, precision=lax.Precision.HIGHEST)
        cnt[...] += lax.dot_general(
            onehot, ones_col, (((0,), (0,)), ((), ())),
            preferred_element_type=jnp.float32, precision=lax.Precision.HIGHEST)

    @pl.when(i == _NBLK - 1)
    def _():
        m1 = pooled1[...] / jnp.maximum(cnt1[...], 1.0)
        m2 = pooled2[...] / jnp.maximum(cnt2[...], 1.0)
        x1 = _lrelu(jnp.dot(m1, p1W[...], preferred_element_type=jnp.float32)
                    + p1b[...][None, :], 0.01)
        x2 = _lrelu(jnp.dot(m2, p2W[...], preferred_element_type=jnp.float32)
                    + p2b[...][None, :], 0.01)
        xc = jnp.concatenate([x1, x2], axis=1)
        hc = _lrelu(jnp.dot(xc, f1W[...], preferred_element_type=jnp.float32)
                    + f1b[...][None, :], 0.01)
        h2 = _lrelu(jnp.dot(hc, f2W[...], preferred_element_type=jnp.float32)
                    + f2b[...][None, :], 0.01)
        out[...] = (jnp.dot(h2, oW[...], preferred_element_type=jnp.float32)
                    + ob[...][None, :])


def _head(acc1, acc2, bat1, bat2, b1g, b2g, p1W, p1b, p2W, p2b,
          f1W, f1b, f2W, f2b, oW, ob):
    full = lambda shape: pl.BlockSpec(shape, lambda i: tuple(0 for _ in shape))
    return pl.pallas_call(
        _head_body,
        grid=(_NBLK,),
        in_specs=[
            pl.BlockSpec((2, _RBLK, _ACCW), lambda i: (0, i, 0)),
            pl.BlockSpec((2, _RBLK, _ACCW), lambda i: (0, i, 0)),
            pl.BlockSpec((1, 1, _RBLK), lambda i: (i, 0, 0)),
            pl.BlockSpec((1, 1, _RBLK), lambda i: (i, 0, 0)),
            full((_HC,)), full((_HC,)),
            full((_HC, _HC)), full((_HC,)),
            full((_HC, _HC)), full((_HC,)),
            full((2 * _HC, _HC)), full((_HC,)),
            full((_HC, 16)), full((16,)),
            full((16, 1)), full((1,)),
        ],
        out_specs=pl.BlockSpec((_B, 1), lambda i: (0, 0)),
        out_shape=jax.ShapeDtypeStruct((_B, 1), jnp.float32),
        scratch_shapes=[
            pltpu.VMEM((_B, _HC), jnp.float32),
            pltpu.VMEM((_B, _HC), jnp.float32),
            pltpu.VMEM((_B, 1), jnp.float32),
            pltpu.VMEM((_B, 1), jnp.float32),
        ],
    )(acc1, acc2, bat1, bat2, b1g, b2g, p1W, p1b, p2W, p2b,
      f1W, f1b, f2W, f2b, oW, ob)


def kernel(ch1_x, ch1_edge_index, ch1_batch, ch2_x, ch2_edge_index, ch2_batch,
           W1l, W1r, att1, b1g, p1W, p1b, W2l, W2r, att2, b2g, p2W, p2b,
           f1W, f1b, f2W, f2b, oW, ob):
    f32 = jnp.float32
    x1p = jnp.pad(ch1_x.astype(f32), ((0, _TBL - _N), (0, 0)))
    x2p = jnp.pad(ch2_x.astype(f32), ((0, _TBL - _N), (0, 0)))

    xl1, xr1, xl2, xr2 = _project(x1p, x2p, W1l, W1r, W2l, W2r)

    def prep_edges(ei):
        src = jnp.concatenate(
            [ei[0].astype(jnp.int32), jnp.zeros((_EPAD - _E,), jnp.int32)])
        dst = jnp.concatenate(
            [ei[1].astype(jnp.int32), jnp.full((_EPAD - _E,), _N, jnp.int32)])
        return (src.reshape(_EPAD // _CHUNK, _CHUNK),
                dst.reshape(_EPAD // _CHUNK, _CHUNK))

    src1, dst1 = prep_edges(ch1_edge_index)
    src2, dst2 = prep_edges(ch2_edge_index)

    acc1 = _edge_pass(xl1, xr1, src1, dst1, att1.reshape(_HC).astype(f32))
    acc2 = _edge_pass(xl2, xr2, src2, dst2, att2.reshape(_HC).astype(f32))

    bat1 = ch1_batch.astype(jnp.int32).reshape(_NBLK, 1, _RBLK)
    bat2 = ch2_batch.astype(jnp.int32).reshape(_NBLK, 1, _RBLK)

    return _head(acc1, acc2, bat1, bat2, b1g, b2g, p1W, p1b, p2W, p2b,
                 f1W, f1b, f2W, f2b, oW, ob)


# speedup vs baseline: 1.2682x; 1.0231x over previous
import dataclasses
import functools

import jax
import jax.numpy as jnp
from jax import lax
from jax.experimental import pallas as pl
from jax.experimental.pallas import tpu as pltpu
from jax.experimental.pallas import tpu_sc as plsc

_N = 10000
_E = 320000
_D = 128
_H = 4
_C = 32
_B = 64
_HC = _H * _C

_NTILES = 32
_CHUNK = 48
_SUPER = 6
_NSUP = -(-_E // (_NTILES * _CHUNK * _SUPER))
_NCHUNK = _NSUP * _SUPER
_NCHUNK0 = _NCHUNK + 24
_NCHUNK1 = 2 * _NCHUNK - _NCHUNK0
_EPT0 = _NCHUNK0 * _CHUNK
_EPT1 = _NCHUNK1 * _CHUNK
_EPAD = 16 * (_EPT0 + _EPT1)
_SLEN = _SUPER * _CHUNK

_ACCW = 144
_ROWS_PER_TILE = (-(-(_N + 1) // 16) + 7) // 8 * 8
_NROWS = _ROWS_PER_TILE * 16
_TBL = _NROWS


def _lrelu(x, s):
    return jnp.maximum(x, s * x)


def _proj_body(x1, x2, w1l, w1r, w2l, w2r, o1l, o1r, o2l, o2r):
    o1l[...] = jnp.dot(x1[...], w1l[...], preferred_element_type=jnp.float32)
    o1r[...] = jnp.dot(x1[...], w1r[...], preferred_element_type=jnp.float32)
    o2l[...] = jnp.dot(x2[...], w2l[...], preferred_element_type=jnp.float32)
    o2r[...] = jnp.dot(x2[...], w2r[...], preferred_element_type=jnp.float32)


def _project(x1p, x2p, W1l, W1r, W2l, W2r):
    sh = jax.ShapeDtypeStruct((_TBL, _HC), jnp.float32)
    return pl.pallas_call(
        _proj_body,
        out_shape=(sh, sh, sh, sh),
    )(x1p, x2p, W1l, W1r, W2l, W2r)


_LOG2E = 1.4426950408889634
_LN2 = 0.6931471805599453
_RND = 12582912.0


def _exp16(p):
    t = p * _LOG2E
    kf = (t + _RND) - _RND
    g = (t - kf) * _LN2
    r = 1.0 / 5040.0
    for c in (1.0 / 720.0, 1.0 / 120.0, 1.0 / 24.0, 1.0 / 6.0, 0.5, 1.0, 1.0):
        r = r * g + c
    ki = kf.astype(jnp.int32)
    bits = plsc.bitcast(r, jnp.int32) + (ki << 23)
    return plsc.bitcast(bits, jnp.float32)


def _vgather(v, idx):
    return lax.gather(
        v, idx[:, None],
        lax.GatherDimensionNumbers(offset_dims=(), collapsed_slice_dims=(0,),
                                   start_index_map=(0,)),
        slice_sizes=(1,), mode=lax.GatherScatterMode.PROMISE_IN_BOUNDS)


def _edge_body(xl_hbm, xr_hbm, src_hbm, dst_hbm, att_hbm, out_hbm,
               src_sb, dst_sb, xl_v0, xr_v0, xl_v1, xr_v1,
               row_v0, row_v1, dvs_v0, dvs_v1, att_v, acc_sh,
               sem0, sem1, ssem0, ssem1):
    row_v = row_v0
    cid = lax.axis_index("c")
    sid = lax.axis_index("s")
    gtid = cid * 16 + sid

    pltpu.sync_copy(att_hbm, att_v)
    att_regs = [(att_v[pl.ds(h * 32, 16)], att_v[pl.ds(h * 32 + 16, 16)])
                for h in range(_H)]
    lane = lax.iota(jnp.int32, 16)
    zero16 = jnp.zeros((16,), jnp.float32)

    @pl.loop(0, _CHUNK)
    def _(r):
        for c in range(_ACCW // 16):
            row_v[r, pl.ds(c * 16, 16)] = zero16

    rbase = sid * _ROWS_PER_TILE
    done = 0
    while done < _ROWS_PER_TILE:
        sz = min(_CHUNK, _ROWS_PER_TILE - done)
        pltpu.sync_copy(row_v.at[pl.ds(0, sz)],
                        acc_sh.at[pl.ds(rbase + done, sz)])
        done += sz
    plsc.subcore_barrier()

    nch2 = jnp.where(cid == 0, _NCHUNK0 // 2, _NCHUNK1 // 2)
    nchunk = nch2 * 2
    cbase = jnp.where(cid == 0, sid * _NCHUNK0,
                      16 * _NCHUNK0 + sid * _NCHUNK1)
    bufs = ((xl_v0, xr_v0, sem0, row_v0, dvs_v0, ssem0),
            (xl_v1, xr_v1, sem1, row_v1, dvs_v1, ssem1))

    def load_super(s):
        off = cbase + s * _SUPER
        pltpu.sync_copy(src_hbm.at[pl.ds(off, _SUPER)], src_sb)
        pltpu.sync_copy(dst_hbm.at[pl.ds(off, _SUPER)], dst_sb)

    def start_gather(ci, b):
        xlv, xrv, sem = bufs[b][:3]
        ciw = lax.rem(ci, _SUPER)
        pltpu.async_copy(xl_hbm.at[src_sb.at[ciw]], xlv, sem)
        pltpu.async_copy(xr_hbm.at[dst_sb.at[ciw]], xrv, sem)

    def wait_gather(b):
        xlv, xrv, sem = bufs[b][:3]
        pltpu.make_async_copy(xl_hbm.at[src_sb.at[0]], xlv, sem).wait()
        pltpu.make_async_copy(xr_hbm.at[dst_sb.at[0]], xrv, sem).wait()

    def wait_scatter(b):
        _, _, _, rv, dvs, ssem = bufs[b]
        pltpu.make_async_copy(rv, acc_sh.at[dvs], ssem).wait()

    def stage_dvs(ci, b):
        dvs = bufs[b][4]
        ciw = lax.rem(ci, _SUPER)
        for c in range(_CHUNK // 16):
            dvs[pl.ds(c * 16, 16)] = dst_sb[ciw, pl.ds(c * 16, 16)]

    def compute_scatter(b):
        xlv, xrv, sem, rv, dvs, ssem = bufs[b]

        @plsc.parallel_loop(0, _CHUNK, unroll=2)
        def _(i):
            den = zero16
            for h in range(_H):
                a0 = xlv[i, pl.ds(h * 32, 16)]
                a1 = xlv[i, pl.ds(h * 32 + 16, 16)]
                b0 = xrv[i, pl.ds(h * 32, 16)]
                b1 = xrv[i, pl.ds(h * 32 + 16, 16)]
                s0 = a0 + b0
                s1 = a1 + b1
                e0 = jnp.maximum(s0, 0.2 * s0)
                e1 = jnp.maximum(s1, 0.2 * s1)
                p = e0 * att_regs[h][0] + e1 * att_regs[h][1]
                for sh in (8, 4, 2, 1):
                    p = p + _vgather(p, lane ^ sh)
                ex = jnp.exp(p)
                rv[i, pl.ds(h * 32, 16)] = ex * a0
                rv[i, pl.ds(h * 32 + 16, 16)] = ex * a1
                den = jnp.where(lane == h, ex, den)
            rv[i, pl.ds(_HC, 16)] = den

        pltpu.async_copy(rv, acc_sh.at[dvs], ssem, add=True)

    load_super(0)
    start_gather(0, 0)

    @pl.loop(0, nch2)
    def _(j):
        c0 = 2 * j
        boundary = lax.rem(c0 + 2, _SUPER) == 0
        start_gather(c0 + 1, 1)
        wait_gather(0)

        @pl.when(j > 0)
        def _():
            wait_scatter(0)

        stage_dvs(c0, 0)
        compute_scatter(0)

        @pl.when(jnp.logical_and(jnp.logical_not(boundary),
                                 c0 + 2 < nchunk))
        def _():
            start_gather(c0 + 2, 0)

        wait_gather(1)

        @pl.when(j > 0)
        def _():
            wait_scatter(1)

        stage_dvs(c0 + 1, 1)

        @pl.when(jnp.logical_and(boundary, c0 + 2 < nchunk))
        def _():
            load_super((c0 + 2) // _SUPER)
            start_gather(c0 + 2, 0)

        compute_scatter(1)

    wait_scatter(0)
    wait_scatter(1)
    plsc.subcore_barrier()

    done = 0
    while done < _ROWS_PER_TILE:
        sz = min(_CHUNK, _ROWS_PER_TILE - done)
        pltpu.sync_copy(acc_sh.at[pl.ds(rbase + done, sz)],
                        out_hbm.at[cid, pl.ds(rbase + done, sz)])
        done += sz


def _edge_pass(xl, xr, src, dst, att_flat):
    mesh = plsc.VectorSubcoreMesh(core_axis_name="c", subcore_axis_name="s")
    cp = pltpu.CompilerParams()
    for f, v in (("needs_layout_passes", False),
                 ("use_tc_tiling_on_sc", False)):
        if f in pltpu.CompilerParams.__dataclass_fields__:
            cp = dataclasses.replace(cp, **{f: v})
    k = functools.partial(
        pl.kernel,
        mesh=mesh,
        compiler_params=cp,
        out_type=jax.ShapeDtypeStruct((2, _NROWS, _ACCW), jnp.float32),
        scratch_types=[
            pltpu.VMEM((_SUPER, _CHUNK), jnp.int32),
            pltpu.VMEM((_SUPER, _CHUNK), jnp.int32),
            pltpu.VMEM((_CHUNK, _D), jnp.float32),
            pltpu.VMEM((_CHUNK, _D), jnp.float32),
            pltpu.VMEM((_CHUNK, _D), jnp.float32),
            pltpu.VMEM((_CHUNK, _D), jnp.float32),
            pltpu.VMEM((_CHUNK, _ACCW), jnp.float32),
            pltpu.VMEM((_CHUNK, _ACCW), jnp.float32),
            pltpu.VMEM((_CHUNK,), jnp.int32),
            pltpu.VMEM((_CHUNK,), jnp.int32),
            pltpu.VMEM((_D,), jnp.float32),
            pltpu.VMEM_SHARED((_NROWS, _ACCW), jnp.float32),
            pltpu.SemaphoreType.DMA,
            pltpu.SemaphoreType.DMA,
            pltpu.SemaphoreType.DMA,
            pltpu.SemaphoreType.DMA,
        ],
    )(_edge_body)
    return k(xl, xr, src, dst, att_flat)


_RBLK = 1000
_NBLK = _N // _RBLK


def _head_body(acc1, acc2, bat1, bat2, b1g, b2g, p1W, p1b, p2W, p2b,
               f1W, f1b, f2W, f2b, oW, ob, out,
               pooled1, pooled2, cnt1, cnt2):
    i = pl.program_id(0)

    @pl.when(i == 0)
    def _():
        pooled1[...] = jnp.zeros_like(pooled1)
        pooled2[...] = jnp.zeros_like(pooled2)
        cnt1[...] = jnp.zeros_like(cnt1)
        cnt2[...] = jnp.zeros_like(cnt2)

    iota_b = lax.broadcasted_iota(jnp.int32, (1, _B), 1)
    ones_col = jnp.ones((_RBLK, 1), jnp.float32)

    for acc, bat, bg, pooled, cnt in (
            (acc1, bat1, b1g, pooled1, cnt1),
            (acc2, bat2, b2g, pooled2, cnt2)):
        blk = acc[...]
        comb = blk[0] + blk[1]
        feats = []
        for h in range(_H):
            d = comb[:, 128 + h][:, None]
            feats.append(comb[:, h * 32:(h + 1) * 32] / (d + 1e-16))
        feat = jnp.concatenate(feats, axis=1) + bg[...][None, :]
        feat = _lrelu(feat, 0.01)
        onehot = (bat[...].reshape(_RBLK, 1) == iota_b).astype(jnp.float32)
        pooled[...] += lax.dot_general(
            onehot, feat, (((0,), (0,)), ((), ())),
            preferred_element_type=jnp.float32, precision=lax.Precision.HIGHEST)
        cnt[...] += lax.dot_general(
            onehot, ones_col, (((0,), (0,)), ((), ())),
            preferred_element_type=jnp.float32, precision=lax.Precision.HIGHEST)

    @pl.when(i == _NBLK - 1)
    def _():
        m1 = pooled1[...] / jnp.maximum(cnt1[...], 1.0)
        m2 = pooled2[...] / jnp.maximum(cnt2[...], 1.0)
        x1 = _lrelu(jnp.dot(m1, p1W[...], preferred_element_type=jnp.float32)
                    + p1b[...][None, :], 0.01)
        x2 = _lrelu(jnp.dot(m2, p2W[...], preferred_element_type=jnp.float32)
                    + p2b[...][None, :], 0.01)
        xc = jnp.concatenate([x1, x2], axis=1)
        hc = _lrelu(jnp.dot(xc, f1W[...], preferred_element_type=jnp.float32)
                    + f1b[...][None, :], 0.01)
        h2 = _lrelu(jnp.dot(hc, f2W[...], preferred_element_type=jnp.float32)
                    + f2b[...][None, :], 0.01)
        out[...] = (jnp.dot(h2, oW[...], preferred_element_type=jnp.float32)
                    + ob[...][None, :])


def _head(acc1, acc2, bat1, bat2, b1g, b2g, p1W, p1b, p2W, p2b,
          f1W, f1b, f2W, f2b, oW, ob):
    full = lambda shape: pl.BlockSpec(shape, lambda i: tuple(0 for _ in shape))
    return pl.pallas_call(
        _head_body,
        grid=(_NBLK,),
        in_specs=[
            pl.BlockSpec((2, _RBLK, _ACCW), lambda i: (0, i, 0)),
            pl.BlockSpec((2, _RBLK, _ACCW), lambda i: (0, i, 0)),
            pl.BlockSpec((1, 1, _RBLK), lambda i: (i, 0, 0)),
            pl.BlockSpec((1, 1, _RBLK), lambda i: (i, 0, 0)),
            full((_HC,)), full((_HC,)),
            full((_HC, _HC)), full((_HC,)),
            full((_HC, _HC)), full((_HC,)),
            full((2 * _HC, _HC)), full((_HC,)),
            full((_HC, 16)), full((16,)),
            full((16, 1)), full((1,)),
        ],
        out_specs=pl.BlockSpec((_B, 1), lambda i: (0, 0)),
        out_shape=jax.ShapeDtypeStruct((_B, 1), jnp.float32),
        scratch_shapes=[
            pltpu.VMEM((_B, _HC), jnp.float32),
            pltpu.VMEM((_B, _HC), jnp.float32),
            pltpu.VMEM((_B, 1), jnp.float32),
            pltpu.VMEM((_B, 1), jnp.float32),
        ],
    )(acc1, acc2, bat1, bat2, b1g, b2g, p1W, p1b, p2W, p2b,
      f1W, f1b, f2W, f2b, oW, ob)


def kernel(ch1_x, ch1_edge_index, ch1_batch, ch2_x, ch2_edge_index, ch2_batch,
           W1l, W1r, att1, b1g, p1W, p1b, W2l, W2r, att2, b2g, p2W, p2b,
           f1W, f1b, f2W, f2b, oW, ob):
    f32 = jnp.float32
    x1p = jnp.pad(ch1_x.astype(f32), ((0, _TBL - _N), (0, 0)))
    x2p = jnp.pad(ch2_x.astype(f32), ((0, _TBL - _N), (0, 0)))

    xl1, xr1, xl2, xr2 = _project(x1p, x2p, W1l, W1r, W2l, W2r)

    def prep_edges(ei):
        src = jnp.concatenate(
            [ei[0].astype(jnp.int32), jnp.zeros((_EPAD - _E,), jnp.int32)])
        dst = jnp.concatenate(
            [ei[1].astype(jnp.int32), jnp.full((_EPAD - _E,), _N, jnp.int32)])
        return (src.reshape(_EPAD // _CHUNK, _CHUNK),
                dst.reshape(_EPAD // _CHUNK, _CHUNK))

    src1, dst1 = prep_edges(ch1_edge_index)
    src2, dst2 = prep_edges(ch2_edge_index)

    acc1 = _edge_pass(xl1, xr1, src1, dst1, att1.reshape(_HC).astype(f32))
    acc2 = _edge_pass(xl2, xr2, src2, dst2, att2.reshape(_HC).astype(f32))

    bat1 = ch1_batch.astype(jnp.int32).reshape(_NBLK, 1, _RBLK)
    bat2 = ch2_batch.astype(jnp.int32).reshape(_NBLK, 1, _RBLK)

    return _head(acc1, acc2, bat1, bat2, b1g, b2g, p1W, p1b, p2W, p2b,
                 f1W, f1b, f2W, f2b, oW, ob)
